# Initial kernel scaffold; baseline (speedup 1.0000x reference)
import jax
import jax.numpy as jnp
from jax.experimental import pallas as pl


def kernel(f, group_idx, W, bn_weight, bn_bias):
    raise NotImplementedError("write your pallas kernel here")


# speedup vs baseline: 19.3497x; 19.3497x over previous
import functools

import jax
import jax.numpy as jnp
from jax import lax
from jax.experimental import pallas as pl
from jax.experimental.pallas import tpu as pltpu
from jax.experimental.pallas import tpu_sc as plsc

B, N, K, C = 8, 2048, 16, 256
CS = 32
NH = 1024
NSL = C // CS
NTASK = B * NSL
NWORK = 32
L = 16


def _mm_body(f_ref, w_ref, o_ref):
    o_ref[...] = lax.dot_general(
        f_ref[...], w_ref[...], (((1,), (1,)), ((), ())),
        preferred_element_type=jnp.float32)


def _matmul(f2d, W):
    blk = 1024
    return pl.pallas_call(
        _mm_body,
        grid=(f2d.shape[0] // blk,),
        in_specs=[
            pl.BlockSpec((blk, C), lambda i: (i, 0)),
            pl.BlockSpec((C, C), lambda i: (0, 0)),
        ],
        out_specs=pl.BlockSpec((blk, C), lambda i: (i, 0)),
        out_shape=jax.ShapeDtypeStruct((f2d.shape[0], C), jnp.float32),
    )(f2d, W)


@functools.partial(
    pl.kernel,
    out_type=[
        jax.ShapeDtypeStruct((B, N, C), jnp.float32),
        jax.ShapeDtypeStruct((B, 2, C), jnp.float32),
    ],
    mesh=plsc.VectorSubcoreMesh(core_axis_name="c", subcore_axis_name="s"),
    compiler_params=pltpu.CompilerParams(use_tc_tiling_on_sc=False),
    scratch_types=[
        pltpu.VMEM((N, CS), jnp.float32),
        pltpu.VMEM((NH, K), jnp.int32),
        pltpu.VMEM((NH, CS), jnp.float32),
        pltpu.VMEM((2, CS), jnp.float32),
    ],
)
def _sc_gather_max(x_hbm, idx_hbm, pooled_hbm, part_hbm,
                   xs_v, idx_v, pooled_v, stats_v):
    wid = lax.axis_index("s") * 2 + lax.axis_index("c")
    for ti in range(NTASK // NWORK):
        t = wid + NWORK * ti
        b = t // NSL
        c0 = (t % NSL) * CS
        pltpu.sync_copy(x_hbm.at[b, :, pl.ds(c0, CS)], xs_v)
        stats = (jnp.zeros((L,), jnp.float32),) * 4
        for h in range(N // NH):
            pltpu.sync_copy(idx_hbm.at[b, pl.ds(h * NH, NH), :], idx_v)

            def body(n, carry, h=h):
                s0, s1, q0, q1 = carry
                iv = idx_v[n, :]
                a = iv[0]
                m0 = xs_v[a, pl.ds(0, L)]
                m1 = xs_v[a, pl.ds(L, L)]
                for j in range(1, K):
                    aj = iv[j]
                    m0 = jnp.maximum(m0, xs_v[aj, pl.ds(0, L)])
                    m1 = jnp.maximum(m1, xs_v[aj, pl.ds(L, L)])
                ng = h * NH + n
                p0 = m0 - xs_v[ng, pl.ds(0, L)]
                p1 = m1 - xs_v[ng, pl.ds(L, L)]
                pooled_v[n, pl.ds(0, L)] = p0
                pooled_v[n, pl.ds(L, L)] = p1
                return (s0 + p0, s1 + p1, q0 + p0 * p0, q1 + p1 * p1)

            stats = lax.fori_loop(0, NH, body, stats)
            pltpu.sync_copy(pooled_v,
                            pooled_hbm.at[b, pl.ds(h * NH, NH), pl.ds(c0, CS)])
        s0, s1, q0, q1 = stats
        stats_v[0, pl.ds(0, L)] = s0
        stats_v[0, pl.ds(L, L)] = s1
        stats_v[1, pl.ds(0, L)] = q0
        stats_v[1, pl.ds(L, L)] = q1
        pltpu.sync_copy(stats_v, part_hbm.at[b, :, pl.ds(c0, CS)])


def _bn_body(pooled_ref, part_ref, w_ref, b_ref, o_ref):
    cnt = float(B * N)
    t = jnp.sum(part_ref[...], axis=0)
    mean = t[0:1, :] / cnt
    meansq = t[1:2, :] / cnt
    var = meansq - mean * mean
    inv = lax.rsqrt(var + 1e-5)
    o_ref[...] = (pooled_ref[...] - mean) * (inv * w_ref[...]) + b_ref[...]


def _batchnorm(pooled2d, partials, bnw2d, bnb2d):
    blk = 1024
    return pl.pallas_call(
        _bn_body,
        grid=(pooled2d.shape[0] // blk,),
        in_specs=[
            pl.BlockSpec((blk, C), lambda i: (i, 0)),
            pl.BlockSpec((B, 2, C), lambda i: (0, 0, 0)),
            pl.BlockSpec((1, C), lambda i: (0, 0)),
            pl.BlockSpec((1, C), lambda i: (0, 0)),
        ],
        out_specs=pl.BlockSpec((blk, C), lambda i: (i, 0)),
        out_shape=jax.ShapeDtypeStruct(pooled2d.shape, jnp.float32),
    )(pooled2d, partials, bnw2d, bnb2d)


def kernel(f, group_idx, W, bn_weight, bn_bias):
    x = _matmul(f.reshape(B * N, C), W)
    pooled, partials = _sc_gather_max(x.reshape(B, N, C), group_idx)
    out = _batchnorm(pooled.reshape(B * N, C), partials,
                     bn_weight.reshape(1, C), bn_bias.reshape(1, C))
    return out.reshape(B, N, C)
